# baseline (speedup 1.0000x reference)
import functools

import jax
import jax.numpy as jnp
from jax import lax
from jax.experimental import pallas as pl
from jax.experimental.pallas import tpu as pltpu
from jax.experimental.pallas import tpu_sc as plsc

_N = 100000
_D = 128
_S = 64
_K = 8
_SK = _S * _K
_NC = 2
_NS = 16
_NW = _NC * _NS
_CHUNK = 80
_TCBLK = 3072
_NTC = 46080
_BASE = _NTC
_NSC = _N - _NTC
_NCHUNKS = _NSC // _CHUNK
_KEXTRA = _NCHUNKS % _NW
_KFULL = _NCHUNKS // _NW
_ZROWS = _SK // _NS


def _sc_body(x_hbm, b_hbm, out_hbm, xb0, xb1, bb0, bb1, ib0, ib1, zbuf, fbuf,
             shared, sx0, sb0, sx1, sb1):
    c = lax.axis_index("c")
    s = lax.axis_index("s")
    wid = c * _NS + s
    iota = lax.broadcasted_iota(jnp.int32, (16,), 0)
    spread = jnp.bitwise_and(iota, _K - 1)
    zv = jnp.zeros((16,), jnp.float32)

    def zrow(i, carry):
        for j8 in range(_D // 16):
            zbuf[i, pl.ds(j8 * 16, 16)] = zv
        return carry

    lax.fori_loop(0, _ZROWS, zrow, 0)
    pltpu.sync_copy(zbuf, shared.at[pl.ds(s * _ZROWS, _ZROWS)])

    plsc.subcore_barrier()

    nk = jnp.where(wid < _KEXTRA, _KFULL + 1, _KFULL)

    pltpu.async_copy(x_hbm.at[pl.ds(_BASE + wid * _CHUNK, _CHUNK)], xb0, sx0)
    pltpu.async_copy(b_hbm.at[wid], bb0, sb0)

    def turn(k, xb_cur, bb_cur, ib_cur, sx_cur, sb_cur, xb_nxt, bb_nxt, sx_nxt,
             sb_nxt):
        @pl.when(k < nk)
        def _():
            @pl.when(k + 1 < nk)
            def _issue_next():
                ch = (k + 1) * _NW + wid
                pltpu.async_copy(x_hbm.at[pl.ds(_BASE + ch * _CHUNK, _CHUNK)],
                                 xb_nxt, sx_nxt)
                pltpu.async_copy(b_hbm.at[ch], bb_nxt, sb_nxt)

            pltpu.make_async_copy(
                x_hbm.at[pl.ds(0, _CHUNK)], xb_cur, sx_cur).wait()
            pltpu.make_async_copy(b_hbm.at[0], bb_cur, sb_cur).wait()
            for m in range(_CHUNK // 16):
                bvec = bb_cur[0, pl.ds(m * 16, 16)]
                ib_cur[0, pl.ds(m * 16, 16)] = bvec * _K + spread
            pltpu.sync_copy(xb_cur, shared.at[ib_cur.at[0]], add=True)

    def body(g, carry):
        turn(2 * g, xb0, bb0, ib0, sx0, sb0, xb1, bb1, sx1, sb1)
        turn(2 * g + 1, xb1, bb1, ib1, sx1, sb1, xb0, bb0, sx0, sb0)
        return carry

    lax.fori_loop(0, (_KFULL + 2) // 2, body, 0)

    plsc.subcore_barrier()

    pltpu.sync_copy(shared.at[pl.ds(s * _ZROWS, _ZROWS)], zbuf)
    for st in range(_ZROWS // _K):
        for j8 in range(_D // 16):
            acc16 = zbuf[st * _K, pl.ds(j8 * 16, 16)]
            for kk in range(1, _K):
                acc16 = acc16 + zbuf[st * _K + kk, pl.ds(j8 * 16, 16)]
            fbuf[st, pl.ds(j8 * 16, 16)] = acc16
    pltpu.sync_copy(fbuf, out_hbm.at[c, pl.ds(s * (_ZROWS // _K), _ZROWS // _K)])


_sc_segment_sum = functools.partial(
    pl.kernel,
    out_type=jax.ShapeDtypeStruct((_NC, _S, _D), jnp.float32),
    mesh=plsc.VectorSubcoreMesh(
        core_axis_name="c", subcore_axis_name="s",
        num_cores=_NC, num_subcores=_NS,
    ),
    scratch_types=[
        pltpu.VMEM((_CHUNK, _D), jnp.float32),
        pltpu.VMEM((_CHUNK, _D), jnp.float32),
        pltpu.VMEM((1, _CHUNK), jnp.int32),
        pltpu.VMEM((1, _CHUNK), jnp.int32),
        pltpu.VMEM((1, _CHUNK), jnp.int32),
        pltpu.VMEM((1, _CHUNK), jnp.int32),
        pltpu.VMEM((_ZROWS, _D), jnp.float32),
        pltpu.VMEM((_ZROWS // _K, _D), jnp.float32),
        pltpu.VMEM_SHARED((_SK, _D), jnp.float32),
        pltpu.SemaphoreType.DMA,
        pltpu.SemaphoreType.DMA,
        pltpu.SemaphoreType.DMA,
        pltpu.SemaphoreType.DMA,
    ],
)(_sc_body)


def _tc_body(x_ref, b_ref, o_ref):
    i = pl.program_id(0)

    @pl.when(i == 0)
    def _init():
        o_ref[...] = jnp.zeros_like(o_ref)

    ids = b_ref[...]
    oh = (ids[None, :] ==
          lax.broadcasted_iota(jnp.int32, (_S, _TCBLK), 0)
          ).astype(jnp.float32)
    o_ref[...] += jnp.dot(oh, x_ref[...],
                          preferred_element_type=jnp.float32)


_tc_partial = pl.pallas_call(
    _tc_body,
    grid=(_NTC // _TCBLK,),
    in_specs=[
        pl.BlockSpec((_TCBLK, _D), lambda i: (i, 0)),
        pl.BlockSpec((_TCBLK,), lambda i: (i,)),
    ],
    out_specs=pl.BlockSpec((_S, _D), lambda i: (0, 0)),
    out_shape=jax.ShapeDtypeStruct((_S, _D), jnp.float32),
)


def _readout_body(f_ref, t_ref, w_ref, o_ref):
    w_row = w_ref[...].reshape(1, _D)
    folded = f_ref[0] + f_ref[1]
    o_ref[...] = jnp.sum((folded + t_ref[...]) * w_row, axis=1)


_readout = pl.pallas_call(
    _readout_body,
    out_shape=jax.ShapeDtypeStruct((_S,), jnp.float32),
)


def kernel(x, batch, W):
    b32 = batch.astype(jnp.int32)
    bsc = b32[_BASE:].reshape(_NCHUNKS, 1, _CHUNK)
    feat = _sc_segment_sum(x, bsc)
    tcp = _tc_partial(x, b32[:_NTC])
    return _readout(feat, tcp, W.reshape(_D))
